# baseline (speedup 1.0000x reference)
import functools

import jax
import jax.numpy as jnp
from jax.experimental import pallas as pl
from jax.experimental.pallas import tpu as pltpu


def _body(a_ref, xt_ref, m0_ref, m12_ref, bias_ref, y_ref, *, rows, nb, grp):
    i = pl.program_id(0)
    a_bf = a_ref[...].astype(jnp.bfloat16)
    x1t = jax.lax.dot_general(
        xt_ref[...], a_bf,
        dimension_numbers=(((1,), (1,)), ((), ())),
        preferred_element_type=jnp.float32).astype(jnp.bfloat16)
    xtb = xt_ref[:, pl.ds(i * rows, rows)]
    parts = []
    for j in range(nb):
        yt = jnp.dot(m0_ref[...], xtb[j * grp:(j + 1) * grp, :],
                     preferred_element_type=jnp.float32)
        yt = yt + jnp.dot(m12_ref[...], x1t[j * grp:(j + 1) * grp, :],
                          preferred_element_type=jnp.float32)
        parts.append(yt)
    y_ref[...] = jnp.concatenate(parts, axis=0) + bias_ref[...]


def kernel(x, support, W, b):
    squeeze = x.ndim < 4
    if squeeze:
        x = x[..., None]
    nb, c_in, n, seq = x.shape
    c_out = W.shape[0]
    grp = seq * c_in

    xt = jnp.transpose(x, (0, 3, 1, 2)).reshape(nb * grp, n).astype(jnp.bfloat16)

    wm = W[:, :, 0, 0]
    w0 = wm[:, :c_in]
    w12 = wm[:, c_in:2 * c_in] + wm[:, 2 * c_in:3 * c_in]
    eye = jnp.eye(seq, dtype=jnp.float32)
    m0 = (jnp.kron(w0, eye).reshape(c_out * seq, c_in, seq)
          .transpose(0, 2, 1).reshape(c_out * seq, grp).astype(jnp.bfloat16))
    m12 = (jnp.kron(w12, eye).reshape(c_out * seq, c_in, seq)
           .transpose(0, 2, 1).reshape(c_out * seq, grp).astype(jnp.bfloat16))
    bias = jnp.repeat(b, seq).reshape(1, c_out * seq)
    bias = jnp.tile(bias, (nb, 1)).reshape(nb * c_out * seq, 1)

    rows = 512
    y_t = pl.pallas_call(
        functools.partial(_body, rows=rows, nb=nb, grp=grp),
        grid=(n // rows,),
        in_specs=[
            pl.BlockSpec((rows, n), lambda i: (i, 0)),
            pl.BlockSpec((nb * grp, n), lambda i: (0, 0)),
            pl.BlockSpec((c_out * seq, grp), lambda i: (0, 0)),
            pl.BlockSpec((c_out * seq, grp), lambda i: (0, 0)),
            pl.BlockSpec((nb * c_out * seq, 1), lambda i: (0, 0)),
        ],
        out_specs=pl.BlockSpec((nb * c_out * seq, rows), lambda i: (0, i)),
        out_shape=jax.ShapeDtypeStruct((nb * c_out * seq, n), jnp.float32),
        compiler_params=pltpu.CompilerParams(
            dimension_semantics=("arbitrary",),
            vmem_limit_bytes=100 * 1024 * 1024,
        ),
    )(support, xt, m0, m12, bias)

    y = y_t.reshape(nb, c_out, seq, n).transpose(0, 1, 3, 2)
    if squeeze:
        y = y[..., 0]
    return y


# speedup vs baseline: 2.4359x; 2.4359x over previous
import functools

import jax
import jax.numpy as jnp
from jax.experimental import pallas as pl
from jax.experimental.pallas import tpu as pltpu


def _body(a_ref, xt_ref, m0_ref, m12_ref, bias_ref, y_ref, *, rows, nb, grp):
    i = pl.program_id(0)
    a_bf = a_ref[...].astype(jnp.bfloat16)
    x1t = jax.lax.dot_general(
        xt_ref[...], a_bf,
        dimension_numbers=(((1,), (1,)), ((), ())),
        preferred_element_type=jnp.float32).astype(jnp.bfloat16)
    xtb = xt_ref[:, pl.ds(i * rows, rows)]
    parts = []
    for j in range(nb):
        yt = jnp.dot(m0_ref[...], xtb[j * grp:(j + 1) * grp, :],
                     preferred_element_type=jnp.float32)
        yt = yt + jnp.dot(m12_ref[...], x1t[j * grp:(j + 1) * grp, :],
                          preferred_element_type=jnp.float32)
        parts.append(yt)
    y_ref[...] = jnp.concatenate(parts, axis=0) + bias_ref[...]


def kernel(x, support, W, b):
    squeeze = x.ndim < 4
    if squeeze:
        x = x[..., None]
    nb, c_in, n, seq = x.shape
    c_out = W.shape[0]
    grp = seq * c_in

    xt = jnp.transpose(x, (0, 3, 1, 2)).reshape(nb * grp, n).astype(jnp.bfloat16)

    wm = W[:, :, 0, 0]
    w0 = wm[:, :c_in]
    w12 = wm[:, c_in:2 * c_in] + wm[:, 2 * c_in:3 * c_in]
    eye = jnp.eye(seq, dtype=jnp.float32)
    m0 = jnp.kron(eye, w0).astype(jnp.bfloat16)
    m12 = jnp.kron(eye, w12).astype(jnp.bfloat16)
    bias = jnp.tile(b, nb * seq).reshape(nb * seq * c_out, 1)

    rows = 512
    y_t = pl.pallas_call(
        functools.partial(_body, rows=rows, nb=nb, grp=grp),
        grid=(n // rows,),
        in_specs=[
            pl.BlockSpec((rows, n), lambda i: (i, 0)),
            pl.BlockSpec((nb * grp, n), lambda i: (0, 0)),
            pl.BlockSpec((seq * c_out, grp), lambda i: (0, 0)),
            pl.BlockSpec((seq * c_out, grp), lambda i: (0, 0)),
            pl.BlockSpec((nb * seq * c_out, 1), lambda i: (0, 0)),
        ],
        out_specs=pl.BlockSpec((nb * seq * c_out, rows), lambda i: (0, i)),
        out_shape=jax.ShapeDtypeStruct((nb * seq * c_out, n), jnp.float32),
        compiler_params=pltpu.CompilerParams(
            dimension_semantics=("arbitrary",),
            vmem_limit_bytes=100 * 1024 * 1024,
        ),
    )(support, xt, m0, m12, bias)

    y = y_t.reshape(nb, seq, c_out, n).transpose(0, 2, 3, 1)
    if squeeze:
        y = y[..., 0]
    return y
